# baseline (speedup 1.0000x reference)
import functools

import jax
import jax.numpy as jnp
from jax import lax
from jax.experimental import pallas as pl
from jax.experimental.pallas import tpu as pltpu
from jax.experimental.pallas import tpu_sc as plsc

N = 10000
F = 128
NP = 10112
E = 320000
NCORE = 2
NSUB = 16
NW = NCORE * NSUB
CHUNK = 96
NBUF = 3
SUP = 16
WIN = SUP
NSUP = 14
NCH = NSUP * SUP
EPT = NCH * CHUNK
EP = NSUB * EPT
ROWS_PER_TILE = NP // NSUB
DUMMY = NP - 1
BLK = 1264
GRID = NP // BLK


@functools.cache
def _sc_kernels():
    mesh = plsc.VectorSubcoreMesh(
        core_axis_name="c", subcore_axis_name="s",
        num_cores=NCORE, num_subcores=NSUB,
    )

    @functools.partial(
        pl.kernel,
        out_type=jax.ShapeDtypeStruct((NCORE * NP, F), jnp.float32),
        mesh=mesh,
        scratch_types=[
            pltpu.VMEM((SUP, CHUNK), jnp.int32),
            pltpu.VMEM((SUP, CHUNK), jnp.int32),
            pltpu.VMEM((CHUNK, F), jnp.float32),
            pltpu.SemaphoreType.DMA,
            pltpu.SemaphoreType.DMA,
            pltpu.VMEM_SHARED((NP, F), jnp.float32),
        ],
    )
    def degree_sc(cidx_hbm, ones_hbm, zeros_hbm, cnt_hbm,
                  idx0, idx1, ones_v, semA, semB, acc_sh):
        c = lax.axis_index("c")
        s = lax.axis_index("s")
        w = c * NSUB + s
        bufs = ((idx0, semA), (idx1, semB))
        cur = pltpu.async_copy(cidx_hbm.at[w, 0], idx0, semA)
        pltpu.sync_copy(ones_hbm, ones_v)
        pltpu.sync_copy(zeros_hbm, acc_sh.at[pl.ds(s * ROWS_PER_TILE, ROWS_PER_TILE)])
        plsc.subcore_barrier()
        for k in range(NSUP):
            buf, _ = bufs[k % 2]
            nbuf, nsem = bufs[(k + 1) % 2]
            nxt = None
            if k + 1 < NSUP:
                nxt = pltpu.async_copy(cidx_hbm.at[w, k + 1], nbuf, nsem)
            cur.wait()

            def body(j, carry, buf=buf):
                pltpu.sync_copy(ones_v, acc_sh.at[buf.at[j]], add=True)
                return carry

            lax.fori_loop(0, SUP, body, 0)
            cur = nxt
        plsc.subcore_barrier()
        off = c * NP + s * ROWS_PER_TILE
        pltpu.sync_copy(
            acc_sh.at[pl.ds(s * ROWS_PER_TILE, ROWS_PER_TILE)],
            cnt_hbm.at[pl.ds(off, ROWS_PER_TILE)],
        )

    @functools.partial(
        pl.kernel,
        out_type=jax.ShapeDtypeStruct((NCORE * NP, F), jnp.float32),
        mesh=mesh,
        scratch_types=[
            pltpu.VMEM((WIN, CHUNK), jnp.int32),
            pltpu.VMEM((WIN, CHUNK), jnp.int32),
            pltpu.VMEM((WIN, CHUNK), jnp.int32),
            pltpu.VMEM((WIN, CHUNK), jnp.int32),
            [pltpu.VMEM((CHUNK, F), jnp.float32) for _ in range(NBUF)],
            pltpu.SemaphoreType.DMA,
            pltpu.SemaphoreType.DMA,
            [pltpu.SemaphoreType.DMA for _ in range(NBUF)],
            [pltpu.SemaphoreType.DMA for _ in range(NBUF)],
            pltpu.VMEM_SHARED((NP, F), jnp.float32),
        ],
    )
    def diffuse_sc(xs_hbm, gwin_hbm, swin_hbm, zeros_hbm, tx_hbm,
                   gidx0, gidx1, sidx0, sidx1, rows, semA, semB,
                   gsem, ssem, acc_sh):
        c = lax.axis_index("c")
        s = lax.axis_index("s")
        w = c * NSUB + s

        def gather(gbuf, j, b):
            return pltpu.async_copy(xs_hbm.at[gbuf.at[j]], rows[b], gsem[b])

        def scatter(sbuf, j, b):
            return pltpu.async_copy(rows[b], acc_sh.at[sbuf.at[j]], ssem[b],
                                    add=True)

        pltpu.sync_copy(zeros_hbm, acc_sh.at[pl.ds(s * ROWS_PER_TILE, ROWS_PER_TILE)])
        plsc.subcore_barrier()

        def body(u, carry):
            wfg0 = pltpu.async_copy(gwin_hbm.at[w, 2 * u], gidx0, semA)
            wfs0 = pltpu.async_copy(swin_hbm.at[w, 2 * u], sidx0, semA)
            wfg1 = pltpu.async_copy(gwin_hbm.at[w, 2 * u + 1], gidx1, semB)
            wfs1 = pltpu.async_copy(swin_hbm.at[w, 2 * u + 1], sidx1, semB)
            wfg0.wait()
            wfs0.wait()
            gd = {}
            sd = {}
            for c in range(2 * SUP + 1):
                win, j = divmod(c, SUP)
                if c == SUP:
                    wfg1.wait()
                    wfs1.wait()
                b = c % NBUF
                if c <= 2 * SUP - 1:
                    if c >= NBUF:
                        sd[c - NBUF].wait()
                    gbuf = gidx0 if win == 0 else gidx1
                    gd[c] = gather(gbuf, j, b)
                if c >= 1:
                    cp = c - 1
                    winp, jp = divmod(cp, SUP)
                    gd[cp].wait()
                    sbuf = sidx0 if winp == 0 else sidx1
                    sd[cp] = scatter(sbuf, jp, cp % NBUF)
            for q in range(2 * SUP - NBUF, 2 * SUP):
                sd[q].wait()
            return carry

        lax.fori_loop(0, NSUP // 2, body, 0)
        plsc.subcore_barrier()
        off = c * NP + s * ROWS_PER_TILE
        pltpu.sync_copy(
            acc_sh.at[pl.ds(s * ROWS_PER_TILE, ROWS_PER_TILE)],
            tx_hbm.at[pl.ds(off, ROWS_PER_TILE)],
        )

    return degree_sc, diffuse_sc


def _inv_wide(cnt_blk):
    return jnp.where(cnt_blk > 0.0, 1.0 / cnt_blk, 0.0)


def _scale_tc(x_ref, cnt_ref, xs_ref):
    x = x_ref[...]
    xs_ref[0] = x * _inv_wide(cnt_ref[0])
    xs_ref[1] = x * _inv_wide(cnt_ref[1])


def _gates_tc(x_ref, tx_ref, cnt_ref, wz_ref, bz_ref, wh_ref, bh_ref,
              h_ref, xs_ref):
    x = x_ref[...]
    to = tx_ref[0]
    ti = tx_ref[1]

    def gate(w_ref, b_ref):
        g = jnp.dot(x, w_ref[0], preferred_element_type=jnp.float32)
        g += jnp.dot(to, w_ref[1], preferred_element_type=jnp.float32)
        g += jnp.dot(ti, w_ref[2], preferred_element_type=jnp.float32)
        return g + b_ref[...]

    z = jax.nn.sigmoid(gate(wz_ref, bz_ref))
    ht = jnp.tanh(gate(wh_ref, bh_ref))
    h = (1.0 - z) * ht
    rid = lax.broadcasted_iota(jnp.int32, (BLK, F), 0) + pl.program_id(0) * BLK
    h = jnp.where(rid < N, h, 0.0)
    h_ref[...] = h
    xs_ref[0] = h * _inv_wide(cnt_ref[0])
    xs_ref[1] = h * _inv_wide(cnt_ref[1])


def _scale_call(x_pad, cnt):
    return pl.pallas_call(
        _scale_tc,
        grid=(GRID,),
        in_specs=[
            pl.BlockSpec((BLK, F), lambda i: (i, 0)),
            pl.BlockSpec((2, BLK, F), lambda i: (0, i, 0)),
        ],
        out_specs=pl.BlockSpec((2, BLK, F), lambda i: (0, i, 0)),
        out_shape=jax.ShapeDtypeStruct((2, NP, F), jnp.float32),
    )(x_pad, cnt)


def _gates_call(x_pad, tx, cnt, wz, bz, wh, bh):
    return pl.pallas_call(
        _gates_tc,
        grid=(GRID,),
        in_specs=[
            pl.BlockSpec((BLK, F), lambda i: (i, 0)),
            pl.BlockSpec((2, BLK, F), lambda i: (0, i, 0)),
            pl.BlockSpec((2, BLK, F), lambda i: (0, i, 0)),
            pl.BlockSpec((3, F, F), lambda i: (0, 0, 0)),
            pl.BlockSpec((1, F), lambda i: (0, 0)),
            pl.BlockSpec((3, F, F), lambda i: (0, 0, 0)),
            pl.BlockSpec((1, F), lambda i: (0, 0)),
        ],
        out_specs=[
            pl.BlockSpec((BLK, F), lambda i: (i, 0)),
            pl.BlockSpec((2, BLK, F), lambda i: (0, i, 0)),
        ],
        out_shape=[
            jax.ShapeDtypeStruct((NP, F), jnp.float32),
            jax.ShapeDtypeStruct((2, NP, F), jnp.float32),
        ],
    )(x_pad, tx, cnt, wz, bz, wh, bh)


def _pack_gate(w):
    return jnp.stack([w[0, 0, :F] + w[1, 0, :F], w[0, 1, :F], w[1, 1, :F]])


def kernel(x, edge_index, Wz0, bz0, Wr0, br0, Wh0, bh0,
           Wz1, bz1, Wr1, br1, Wh1, bh1):
    src = edge_index[0].astype(jnp.int32)
    dst = edge_index[1].astype(jnp.int32)
    pad = EP - E
    srcp = jnp.concatenate([src, jnp.full((pad,), DUMMY, jnp.int32)])
    dstp = jnp.concatenate([dst, jnp.full((pad,), DUMMY, jnp.int32)])

    gathf = jnp.stack([srcp, dstp + NP]).reshape(NCORE, NSUB, NCH, CHUNK)
    scatf = jnp.stack([dstp, srcp]).reshape(NCORE, NSUB, NCH, CHUNK)
    gdum = jnp.broadcast_to(
        jnp.array([DUMMY, NP + DUMMY], jnp.int32)[:, None, None, None],
        (NCORE, NSUB, WIN - SUP, CHUNK))
    sdum = jnp.full((NCORE, NSUB, WIN - SUP, CHUNK), DUMMY, jnp.int32)
    gathf = jnp.concatenate([gathf, gdum], axis=2)
    scatf = jnp.concatenate([scatf, sdum], axis=2)
    gathf = gathf.reshape(NW, NCH + WIN - SUP, CHUNK)
    scatf = scatf.reshape(NW, NCH + WIN - SUP, CHUNK)
    gwin = jnp.stack(
        [gathf[:, SUP * k:SUP * k + WIN] for k in range(NSUP)], axis=1)
    swin = jnp.stack(
        [scatf[:, SUP * k:SUP * k + WIN] for k in range(NSUP)], axis=1)
    cidx = jnp.stack([srcp, dstp]).reshape(NW, NSUP, SUP, CHUNK)

    ones128 = jnp.ones((CHUNK, F), jnp.float32)
    zeros128 = jnp.zeros((ROWS_PER_TILE, F), jnp.float32)
    x_pad = jnp.concatenate([x, jnp.zeros((NP - N, F), jnp.float32)])

    degree_sc, diffuse_sc = _sc_kernels()
    cnt = degree_sc(cidx, ones128, zeros128).reshape(NCORE, NP, F)

    wz0 = _pack_gate(Wz0)
    wh0 = _pack_gate(Wh0)
    wz1 = _pack_gate(Wz1)
    wh1 = _pack_gate(Wh1)
    bz0r = bz0.reshape(1, F)
    bh0r = bh0.reshape(1, F)
    bz1r = bz1.reshape(1, F)
    bh1r = bh1.reshape(1, F)

    xs0 = _scale_call(x_pad, cnt)
    tx0 = diffuse_sc(xs0.reshape(NCORE * NP, F), gwin, swin, zeros128)
    h0, xs1 = _gates_call(x_pad, tx0.reshape(NCORE, NP, F), cnt,
                          wz0, bz0r, wh0, bh0r)
    tx1 = diffuse_sc(xs1.reshape(NCORE * NP, F), gwin, swin, zeros128)
    h1, _ = _gates_call(h0, tx1.reshape(NCORE, NP, F), cnt,
                        wz1, bz1r, wh1, bh1r)
    return jnp.stack([h0[:N], h1[:N]])


# speedup vs baseline: 2.7823x; 2.7823x over previous
import functools

import jax
import jax.numpy as jnp
from jax import lax
from jax.experimental import pallas as pl
from jax.experimental.pallas import tpu as pltpu
from jax.experimental.pallas import tpu_sc as plsc

N = 10000
F = 128
NP = 10112
E = 320000
NCORE = 2
NSUB = 16
NW = NCORE * NSUB
CHUNK = 128
NBUF = 2
SUP = 16
WIN = SUP
NSUP = 10
NCH = NSUP * SUP
EPT = NCH * CHUNK
EP = NSUB * EPT
ROWS_PER_TILE = NP // NSUB
DUMMY = NP - 1
BLK = 1264
GRID = NP // BLK


@functools.cache
def _sc_kernels():
    mesh = plsc.VectorSubcoreMesh(
        core_axis_name="c", subcore_axis_name="s",
        num_cores=NCORE, num_subcores=NSUB,
    )

    @functools.partial(
        pl.kernel,
        out_type=jax.ShapeDtypeStruct((NCORE * NP, F), jnp.float32),
        mesh=mesh,
        scratch_types=[
            pltpu.VMEM((SUP, CHUNK), jnp.int32),
            pltpu.VMEM((SUP, CHUNK), jnp.int32),
            pltpu.VMEM((CHUNK, F), jnp.float32),
            pltpu.SemaphoreType.DMA,
            pltpu.SemaphoreType.DMA,
            pltpu.SemaphoreType.DMA,
            pltpu.VMEM_SHARED((NP, F), jnp.float32),
        ],
    )
    def degree_sc(cidx_hbm, ones_hbm, zeros_hbm, cnt_hbm,
                  idx0, idx1, ones_v, semA, semB, ssemD, acc_sh):
        c = lax.axis_index("c")
        s = lax.axis_index("s")
        w = c * NSUB + s
        pltpu.sync_copy(ones_hbm, ones_v)
        pltpu.sync_copy(zeros_hbm, acc_sh.at[pl.ds(s * ROWS_PER_TILE, ROWS_PER_TILE)])
        plsc.subcore_barrier()

        def body(u, carry):
            f0 = pltpu.async_copy(cidx_hbm.at[w, 2 * u], idx0, semA)
            f1 = pltpu.async_copy(cidx_hbm.at[w, 2 * u + 1], idx1, semB)
            sds = []
            f0.wait()
            for j in range(SUP):
                sds.append(pltpu.async_copy(
                    ones_v, acc_sh.at[idx0.at[j]], ssemD, add=True))
            f1.wait()
            for j in range(SUP):
                sds.append(pltpu.async_copy(
                    ones_v, acc_sh.at[idx1.at[j]], ssemD, add=True))
            for d in sds:
                d.wait()
            return carry

        lax.fori_loop(0, NSUP // 2, body, 0)
        plsc.subcore_barrier()
        off = c * NP + s * ROWS_PER_TILE
        pltpu.sync_copy(
            acc_sh.at[pl.ds(s * ROWS_PER_TILE, ROWS_PER_TILE)],
            cnt_hbm.at[pl.ds(off, ROWS_PER_TILE)],
        )

    @functools.partial(
        pl.kernel,
        out_type=jax.ShapeDtypeStruct((NCORE * NP, F), jnp.float32),
        mesh=mesh,
        scratch_types=[
            pltpu.VMEM((WIN, CHUNK), jnp.int32),
            pltpu.VMEM((WIN, CHUNK), jnp.int32),
            pltpu.VMEM((WIN, CHUNK), jnp.int32),
            pltpu.VMEM((WIN, CHUNK), jnp.int32),
            [pltpu.VMEM((CHUNK, F), jnp.float32) for _ in range(NBUF)],
            pltpu.SemaphoreType.DMA,
            pltpu.SemaphoreType.DMA,
            [pltpu.SemaphoreType.DMA for _ in range(NBUF)],
            [pltpu.SemaphoreType.DMA for _ in range(NBUF)],
            pltpu.VMEM_SHARED((NP, F), jnp.float32),
        ],
    )
    def diffuse_sc(xs_hbm, gwin_hbm, swin_hbm, zeros_hbm, tx_hbm,
                   gidx0, gidx1, sidx0, sidx1, rows, semA, semB,
                   gsem, ssem, acc_sh):
        c = lax.axis_index("c")
        s = lax.axis_index("s")
        w = c * NSUB + s

        def gather(gbuf, j, b):
            return pltpu.async_copy(xs_hbm.at[gbuf.at[j]], rows[b], gsem[b])

        def scatter(sbuf, j, b):
            return pltpu.async_copy(rows[b], acc_sh.at[sbuf.at[j]], ssem[b],
                                    add=True)

        pltpu.sync_copy(zeros_hbm, acc_sh.at[pl.ds(s * ROWS_PER_TILE, ROWS_PER_TILE)])
        plsc.subcore_barrier()

        def body(u, carry):
            wfg0 = pltpu.async_copy(gwin_hbm.at[w, 2 * u], gidx0, semA)
            wfs0 = pltpu.async_copy(swin_hbm.at[w, 2 * u], sidx0, semA)
            wfg1 = pltpu.async_copy(gwin_hbm.at[w, 2 * u + 1], gidx1, semB)
            wfs1 = pltpu.async_copy(swin_hbm.at[w, 2 * u + 1], sidx1, semB)
            wfg0.wait()
            wfs0.wait()
            gd = {}
            sd = {}
            for c in range(2 * SUP + 1):
                win, j = divmod(c, SUP)
                if c == SUP:
                    wfg1.wait()
                    wfs1.wait()
                b = c % NBUF
                if c <= 2 * SUP - 1:
                    if c >= NBUF:
                        sd[c - NBUF].wait()
                    gbuf = gidx0 if win == 0 else gidx1
                    gd[c] = gather(gbuf, j, b)
                if c >= 1:
                    cp = c - 1
                    winp, jp = divmod(cp, SUP)
                    gd[cp].wait()
                    sbuf = sidx0 if winp == 0 else sidx1
                    sd[cp] = scatter(sbuf, jp, cp % NBUF)
            for q in range(2 * SUP - NBUF, 2 * SUP):
                sd[q].wait()
            return carry

        lax.fori_loop(0, NSUP // 2, body, 0)
        plsc.subcore_barrier()
        off = c * NP + s * ROWS_PER_TILE
        pltpu.sync_copy(
            acc_sh.at[pl.ds(s * ROWS_PER_TILE, ROWS_PER_TILE)],
            tx_hbm.at[pl.ds(off, ROWS_PER_TILE)],
        )

    return degree_sc, diffuse_sc


def _inv_wide(cnt_blk):
    return jnp.where(cnt_blk > 0.0, 1.0 / cnt_blk, 0.0)


def _scale_tc(x_ref, cnt_ref, xs_ref):
    x = x_ref[...]
    xs_ref[0] = x * _inv_wide(cnt_ref[0])
    xs_ref[1] = x * _inv_wide(cnt_ref[1])


def _gates_tc(x_ref, tx_ref, cnt_ref, wz_ref, bz_ref, wh_ref, bh_ref,
              h_ref, xs_ref):
    x = x_ref[...]
    to = tx_ref[0]
    ti = tx_ref[1]

    def gate(w_ref, b_ref):
        g = jnp.dot(x, w_ref[0], preferred_element_type=jnp.float32)
        g += jnp.dot(to, w_ref[1], preferred_element_type=jnp.float32)
        g += jnp.dot(ti, w_ref[2], preferred_element_type=jnp.float32)
        return g + b_ref[...]

    z = jax.nn.sigmoid(gate(wz_ref, bz_ref))
    ht = jnp.tanh(gate(wh_ref, bh_ref))
    h = (1.0 - z) * ht
    rid = lax.broadcasted_iota(jnp.int32, (BLK, F), 0) + pl.program_id(0) * BLK
    h = jnp.where(rid < N, h, 0.0)
    h_ref[...] = h
    xs_ref[0] = h * _inv_wide(cnt_ref[0])
    xs_ref[1] = h * _inv_wide(cnt_ref[1])


def _scale_call(x_pad, cnt):
    return pl.pallas_call(
        _scale_tc,
        grid=(GRID,),
        in_specs=[
            pl.BlockSpec((BLK, F), lambda i: (i, 0)),
            pl.BlockSpec((2, BLK, F), lambda i: (0, i, 0)),
        ],
        out_specs=pl.BlockSpec((2, BLK, F), lambda i: (0, i, 0)),
        out_shape=jax.ShapeDtypeStruct((2, NP, F), jnp.float32),
    )(x_pad, cnt)


def _gates_call(x_pad, tx, cnt, wz, bz, wh, bh):
    return pl.pallas_call(
        _gates_tc,
        grid=(GRID,),
        in_specs=[
            pl.BlockSpec((BLK, F), lambda i: (i, 0)),
            pl.BlockSpec((2, BLK, F), lambda i: (0, i, 0)),
            pl.BlockSpec((2, BLK, F), lambda i: (0, i, 0)),
            pl.BlockSpec((3, F, F), lambda i: (0, 0, 0)),
            pl.BlockSpec((1, F), lambda i: (0, 0)),
            pl.BlockSpec((3, F, F), lambda i: (0, 0, 0)),
            pl.BlockSpec((1, F), lambda i: (0, 0)),
        ],
        out_specs=[
            pl.BlockSpec((BLK, F), lambda i: (i, 0)),
            pl.BlockSpec((2, BLK, F), lambda i: (0, i, 0)),
        ],
        out_shape=[
            jax.ShapeDtypeStruct((NP, F), jnp.float32),
            jax.ShapeDtypeStruct((2, NP, F), jnp.float32),
        ],
    )(x_pad, tx, cnt, wz, bz, wh, bh)


def _pack_gate(w):
    return jnp.stack([w[0, 0, :F] + w[1, 0, :F], w[0, 1, :F], w[1, 1, :F]])


def kernel(x, edge_index, Wz0, bz0, Wr0, br0, Wh0, bh0,
           Wz1, bz1, Wr1, br1, Wh1, bh1):
    src = edge_index[0].astype(jnp.int32)
    dst = edge_index[1].astype(jnp.int32)
    pad = EP - E
    srcp = jnp.concatenate([src, jnp.full((pad,), DUMMY, jnp.int32)])
    dstp = jnp.concatenate([dst, jnp.full((pad,), DUMMY, jnp.int32)])

    gathf = jnp.stack([srcp, dstp + NP]).reshape(NCORE, NSUB, NCH, CHUNK)
    scatf = jnp.stack([dstp, srcp]).reshape(NCORE, NSUB, NCH, CHUNK)
    gdum = jnp.broadcast_to(
        jnp.array([DUMMY, NP + DUMMY], jnp.int32)[:, None, None, None],
        (NCORE, NSUB, WIN - SUP, CHUNK))
    sdum = jnp.full((NCORE, NSUB, WIN - SUP, CHUNK), DUMMY, jnp.int32)
    gathf = jnp.concatenate([gathf, gdum], axis=2)
    scatf = jnp.concatenate([scatf, sdum], axis=2)
    gathf = gathf.reshape(NW, NCH + WIN - SUP, CHUNK)
    scatf = scatf.reshape(NW, NCH + WIN - SUP, CHUNK)
    gwin = jnp.stack(
        [gathf[:, SUP * k:SUP * k + WIN] for k in range(NSUP)], axis=1)
    swin = jnp.stack(
        [scatf[:, SUP * k:SUP * k + WIN] for k in range(NSUP)], axis=1)
    cidx = jnp.stack([srcp, dstp]).reshape(NW, NSUP, SUP, CHUNK)

    ones128 = jnp.ones((CHUNK, F), jnp.float32)
    zeros128 = jnp.zeros((ROWS_PER_TILE, F), jnp.float32)
    x_pad = jnp.concatenate([x, jnp.zeros((NP - N, F), jnp.float32)])

    degree_sc, diffuse_sc = _sc_kernels()
    cnt = degree_sc(cidx, ones128, zeros128).reshape(NCORE, NP, F)

    wz0 = _pack_gate(Wz0)
    wh0 = _pack_gate(Wh0)
    wz1 = _pack_gate(Wz1)
    wh1 = _pack_gate(Wh1)
    bz0r = bz0.reshape(1, F)
    bh0r = bh0.reshape(1, F)
    bz1r = bz1.reshape(1, F)
    bh1r = bh1.reshape(1, F)

    xs0 = _scale_call(x_pad, cnt)
    tx0 = diffuse_sc(xs0.reshape(NCORE * NP, F), gwin, swin, zeros128)
    h0, xs1 = _gates_call(x_pad, tx0.reshape(NCORE, NP, F), cnt,
                          wz0, bz0r, wh0, bh0r)
    tx1 = diffuse_sc(xs1.reshape(NCORE * NP, F), gwin, swin, zeros128)
    h1, _ = _gates_call(h0, tx1.reshape(NCORE, NP, F), cnt,
                        wz1, bz1r, wh1, bh1r)
    return jnp.stack([h0[:N], h1[:N]])
